# baseline (speedup 1.0000x reference)
import functools

import jax
import jax.numpy as jnp
from jax import lax
from jax.experimental import pallas as pl
from jax.experimental.pallas import tpu as pltpu
from jax.experimental.pallas import tpu_sc as plsc

VOCAB = 1000001
EMB = 64
SEQ = 200
BATCH = 4096
NUM_CORES = 2
NUM_SUBCORES = 16
NW = NUM_CORES * NUM_SUBCORES
BPW = BATCH // NW
LANES = 16
BC = 65536
NBLK = (VOCAB + BC - 1) // BC
VPAD = NBLK * BC


def _tc_tw(table_t, w_row, b2):
    def body(t_ref, w_ref, b_ref, o_ref):
        o_ref[...] = (
            lax.dot_general(w_ref[...], t_ref[...], (((1,), (0,)), ((), ())),
                            preferred_element_type=jnp.float32)
            * (1.0 / SEQ)
            + b_ref[...] * (1.0 / SEQ)
        )

    return pl.pallas_call(
        body,
        grid=(NBLK,),
        in_specs=[
            pl.BlockSpec((EMB, BC), lambda i: (0, i)),
            pl.BlockSpec((1, EMB), lambda i: (0, 0)),
            pl.BlockSpec((1, 1), lambda i: (0, 0)),
        ],
        out_specs=pl.BlockSpec((1, BC), lambda i: (0, i)),
        out_shape=jax.ShapeDtypeStruct((1, VPAD), jnp.float32),
    )(table_t, w_row, b2)


G = 1024
NG = SEQ * BPW // G
NRING = 5


def _sc_gather_sum(tw, x4):
    mesh = plsc.VectorSubcoreMesh(
        core_axis_name="c", subcore_axis_name="s",
        num_cores=NUM_CORES, num_subcores=NUM_SUBCORES)

    @functools.partial(
        pl.kernel,
        out_type=jax.ShapeDtypeStruct((BATCH,), jnp.float32),
        mesh=mesh,
        scratch_types=[
            pltpu.VMEM((NG, G), jnp.int32),
            pltpu.VMEM((BPW,), jnp.float32),
            pltpu.VMEM((NRING, G), jnp.float32),
            [pltpu.SemaphoreType.DMA] * NRING,
        ],
        compiler_params=pltpu.CompilerParams(use_tc_tiling_on_sc=False),
    )
    def k(tw_hbm, x_hbm, out_hbm, idx_v, acc_v, buf_v, sems):
        wid = lax.axis_index("s") * NUM_CORES + lax.axis_index("c")
        base = wid * BPW
        pltpu.sync_copy(x_hbm.at[wid], idx_v)

        for cc in range(BPW // LANES):
            acc_v[pl.ds(cc * LANES, LANES)] = jnp.zeros((LANES,), jnp.float32)

        for u in range(NRING):
            pltpu.async_copy(tw_hbm.at[idx_v.at[u]], buf_v.at[u], sems[u])

        def wait_buf(u):
            pltpu.make_async_copy(
                tw_hbm.at[idx_v.at[0]], buf_v.at[u], sems[u]).wait()

        def accumulate(u):
            for cc in range(G // LANES):
                sl = pl.ds(cc * LANES, LANES)
                plsc.addupdate(acc_v.at[pl.ds((cc % 8) * LANES, LANES)],
                               buf_v[u, sl])

        def body(kk, carry):
            for u in range(NRING):
                j = NRING * kk + u
                wait_buf(u)
                accumulate(u)

                @pl.when(j + NRING <= NG - 1)
                def _():
                    pltpu.async_copy(
                        tw_hbm.at[idx_v.at[j + NRING]], buf_v.at[u], sems[u])
            return carry

        lax.fori_loop(0, NG // NRING - 1, body, None)
        for u in range(NRING):
            wait_buf(u)
            accumulate(u)

        pltpu.sync_copy(acc_v, out_hbm.at[pl.ds(base, BPW)])

    return k(tw, x4)


def kernel(x, table, W, b):
    x32 = x.astype(jnp.int32)
    x4 = x32.reshape(SEQ, NW, BPW).transpose(1, 0, 2).reshape(NW, NG, G)
    tw2d = _tc_tw(table.T, W.reshape(1, EMB), b.reshape(1, 1))
    tw = tw2d.reshape(VPAD)
    return _sc_gather_sum(tw, x4)


# speedup vs baseline: 1.0168x; 1.0168x over previous
import functools

import jax
import jax.numpy as jnp
from jax import lax
from jax.experimental import pallas as pl
from jax.experimental.pallas import tpu as pltpu
from jax.experimental.pallas import tpu_sc as plsc

VOCAB = 1000001
EMB = 64
SEQ = 200
BATCH = 4096
NUM_CORES = 2
NUM_SUBCORES = 16
NW = NUM_CORES * NUM_SUBCORES
BPW = BATCH // NW
LANES = 16
BC = 32768
NBLK = (VOCAB + BC - 1) // BC
VPAD = NBLK * BC


def _tc_tw(table_t, w_row, b2):
    def body(t_ref, w_ref, b_ref, o_ref):
        o_ref[...] = (
            lax.dot_general(w_ref[...], t_ref[...], (((1,), (0,)), ((), ())),
                            preferred_element_type=jnp.float32)
            * (1.0 / SEQ)
            + b_ref[...] * (1.0 / SEQ)
        )

    return pl.pallas_call(
        body,
        grid=(NBLK,),
        in_specs=[
            pl.BlockSpec((EMB, BC), lambda i: (0, i)),
            pl.BlockSpec((1, EMB), lambda i: (0, 0)),
            pl.BlockSpec((1, 1), lambda i: (0, 0)),
        ],
        out_specs=pl.BlockSpec((1, BC), lambda i: (0, i)),
        out_shape=jax.ShapeDtypeStruct((1, VPAD), jnp.float32),
    )(table_t, w_row, b2)


G = 1024
NG = SEQ * BPW // G
NRING = 5


def _sc_gather_sum(tw, x4):
    mesh = plsc.VectorSubcoreMesh(
        core_axis_name="c", subcore_axis_name="s",
        num_cores=NUM_CORES, num_subcores=NUM_SUBCORES)

    @functools.partial(
        pl.kernel,
        out_type=jax.ShapeDtypeStruct((BATCH,), jnp.float32),
        mesh=mesh,
        scratch_types=[
            pltpu.VMEM((NG, G), jnp.int32),
            pltpu.VMEM((BPW,), jnp.float32),
            pltpu.VMEM((NRING, G), jnp.float32),
            [pltpu.SemaphoreType.DMA] * NRING,
        ],
        compiler_params=pltpu.CompilerParams(use_tc_tiling_on_sc=False),
    )
    def k(tw_hbm, x_hbm, out_hbm, idx_v, acc_v, buf_v, sems):
        wid = lax.axis_index("s") * NUM_CORES + lax.axis_index("c")
        base = wid * BPW
        pltpu.sync_copy(x_hbm.at[wid], idx_v)

        for cc in range(BPW // LANES):
            acc_v[pl.ds(cc * LANES, LANES)] = jnp.zeros((LANES,), jnp.float32)

        for u in range(NRING):
            pltpu.async_copy(tw_hbm.at[idx_v.at[u]], buf_v.at[u], sems[u])

        def wait_buf(u):
            pltpu.make_async_copy(
                tw_hbm.at[idx_v.at[0]], buf_v.at[u], sems[u]).wait()

        def accumulate(u):
            for cc in range(G // LANES):
                sl = pl.ds(cc * LANES, LANES)
                plsc.addupdate(acc_v.at[pl.ds((cc % 8) * LANES, LANES)],
                               buf_v[u, sl])

        def body(kk, carry):
            for u in range(NRING):
                j = NRING * kk + u
                wait_buf(u)
                accumulate(u)

                @pl.when(j + NRING <= NG - 1)
                def _():
                    pltpu.async_copy(
                        tw_hbm.at[idx_v.at[j + NRING]], buf_v.at[u], sems[u])
            return carry

        lax.fori_loop(0, NG // NRING - 1, body, None)
        for u in range(NRING):
            wait_buf(u)
            accumulate(u)

        pltpu.sync_copy(acc_v, out_hbm.at[pl.ds(base, BPW)])

    return k(tw, x4)


def kernel(x, table, W, b):
    x32 = x.astype(jnp.int32)
    x4 = x32.reshape(SEQ, NW, BPW).transpose(1, 0, 2).reshape(NW, NG, G)
    tw2d = _tc_tw(table.T, W.reshape(1, EMB), b.reshape(1, 1))
    tw = tw2d.reshape(VPAD)
    return _sc_gather_sum(tw, x4)


# speedup vs baseline: 1.1935x; 1.1737x over previous
import functools

import jax
import jax.numpy as jnp
from jax import lax
from jax.experimental import pallas as pl
from jax.experimental.pallas import tpu as pltpu
from jax.experimental.pallas import tpu_sc as plsc

VOCAB = 1000001
EMB = 64
SEQ = 200
BATCH = 4096
NUM_CORES = 2
NUM_SUBCORES = 16
NW = NUM_CORES * NUM_SUBCORES
BPW = BATCH // NW
LANES = 16
BC = 32768
NBLK = (VOCAB + BC - 1) // BC
VPAD = NBLK * BC


def _tc_tw(table_t, w_row, b2):
    def body(t_ref, w_ref, b_ref, o_ref):
        o_ref[...] = (
            lax.dot_general(w_ref[...], t_ref[...], (((1,), (0,)), ((), ())),
                            preferred_element_type=jnp.float32)
            * (1.0 / SEQ)
            + b_ref[...] * (1.0 / SEQ)
        )

    return pl.pallas_call(
        body,
        grid=(NBLK,),
        in_specs=[
            pl.BlockSpec((EMB, BC), lambda i: (0, i)),
            pl.BlockSpec((1, EMB), lambda i: (0, 0)),
            pl.BlockSpec((1, 1), lambda i: (0, 0)),
        ],
        out_specs=pl.BlockSpec((1, BC), lambda i: (0, i)),
        out_shape=jax.ShapeDtypeStruct((1, VPAD), jnp.float32),
    )(table_t, w_row, b2)


G = 512
NG = SEQ * BPW // G
NRING = 5


def _sc_gather_sum(tw, x4):
    mesh = plsc.VectorSubcoreMesh(
        core_axis_name="c", subcore_axis_name="s",
        num_cores=NUM_CORES, num_subcores=NUM_SUBCORES)

    @functools.partial(
        pl.kernel,
        out_type=jax.ShapeDtypeStruct((BATCH,), jnp.float32),
        mesh=mesh,
        scratch_types=[
            pltpu.VMEM((NG, G), jnp.int32),
            pltpu.VMEM((BPW,), jnp.float32),
            pltpu.VMEM((NRING, G), jnp.float32),
            pltpu.VMEM_SHARED((VPAD,), jnp.float32),
            [pltpu.SemaphoreType.DMA] * NRING,
        ],
        compiler_params=pltpu.CompilerParams(use_tc_tiling_on_sc=False),
    )
    def k(tw_hbm, x_hbm, out_hbm, idx_v, acc_v, buf_v, tw_sp, sems):
        wid = lax.axis_index("s") * NUM_CORES + lax.axis_index("c")
        sid = lax.axis_index("s")
        base = wid * BPW

        twc = VPAD // NUM_SUBCORES
        pltpu.sync_copy(tw_hbm.at[pl.ds(sid * twc, twc)],
                        tw_sp.at[pl.ds(sid * twc, twc)])
        pltpu.sync_copy(x_hbm.at[wid], idx_v)
        plsc.subcore_barrier()

        for cc in range(BPW // LANES):
            acc_v[pl.ds(cc * LANES, LANES)] = jnp.zeros((LANES,), jnp.float32)

        for u in range(NRING):
            pltpu.async_copy(tw_sp.at[idx_v.at[u]], buf_v.at[u], sems[u])

        def wait_buf(u):
            pltpu.make_async_copy(
                tw_sp.at[idx_v.at[0]], buf_v.at[u], sems[u]).wait()

        def accumulate(u):
            for cc in range(G // LANES):
                sl = pl.ds(cc * LANES, LANES)
                plsc.addupdate(acc_v.at[pl.ds((cc % 8) * LANES, LANES)],
                               buf_v[u, sl])

        def body(kk, carry):
            for u in range(NRING):
                j = NRING * kk + u
                wait_buf(u)
                accumulate(u)

                @pl.when(j + NRING <= NG - 1)
                def _():
                    pltpu.async_copy(
                        tw_sp.at[idx_v.at[j + NRING]], buf_v.at[u], sems[u])
            return carry

        lax.fori_loop(0, NG // NRING - 1, body, None)
        for u in range(NRING):
            wait_buf(u)
            accumulate(u)

        pltpu.sync_copy(acc_v, out_hbm.at[pl.ds(base, BPW)])

    return k(tw, x4)


def kernel(x, table, W, b):
    x32 = x.astype(jnp.int32)
    x4 = x32.reshape(SEQ, NW, BPW).transpose(1, 0, 2).reshape(NW, NG, G)
    tw2d = _tc_tw(table.T, W.reshape(1, EMB), b.reshape(1, 1))
    tw = tw2d.reshape(VPAD)
    return _sc_gather_sum(tw, x4)
